# speedup vs baseline: 2.9330x; 2.9330x over previous
import jax
import jax.numpy as jnp
from jax.experimental import pallas as pl

B, M, D = 1024, 2048, 16
BR = 32


def _body(times_ref, mask_ref, aux_ref, markcol_ref, marks_ref,
          tout_ref, mout_ref, marksout_ref):
    m = mask_ref[...]
    cnt = jnp.sum(m.astype(jnp.int32), axis=1, keepdims=True)
    acc = aux_ref[:, 1:2] > 0.5
    canw = acc & (cnt < M)
    safe = jnp.minimum(cnt, M - 1)
    lanes = jax.lax.broadcasted_iota(jnp.int32, (BR, M), 1)
    sel = (lanes == safe) & canw
    tout_ref[...] = jnp.where(sel, aux_ref[:, 0:1], times_ref[...])
    mout_ref[...] = m | sel
    marksout_ref[...] = jnp.where(sel[:, None, :], markcol_ref[...],
                                  marks_ref[...])


def kernel(times, mask, marks, t, mark, accepted):
    marks_t = jnp.transpose(marks, (0, 2, 1))
    aux = jnp.concatenate(
        [t[:, None], accepted.astype(jnp.float32)[:, None],
         jnp.zeros((B, 126), jnp.float32)], axis=1)
    markcol = mark[:, :, None]

    grid = (B // BR,)
    new_times, new_mask, new_marks_t = pl.pallas_call(
        _body,
        grid=grid,
        in_specs=[
            pl.BlockSpec((BR, M), lambda i: (i, 0)),
            pl.BlockSpec((BR, M), lambda i: (i, 0)),
            pl.BlockSpec((BR, 128), lambda i: (i, 0)),
            pl.BlockSpec((BR, D, 1), lambda i: (i, 0, 0)),
            pl.BlockSpec((BR, D, M), lambda i: (i, 0, 0)),
        ],
        out_specs=[
            pl.BlockSpec((BR, M), lambda i: (i, 0)),
            pl.BlockSpec((BR, M), lambda i: (i, 0)),
            pl.BlockSpec((BR, D, M), lambda i: (i, 0, 0)),
        ],
        out_shape=[
            jax.ShapeDtypeStruct((B, M), jnp.float32),
            jax.ShapeDtypeStruct((B, M), jnp.bool_),
            jax.ShapeDtypeStruct((B, D, M), jnp.float32),
        ],
    )(times, mask, aux, markcol, marks_t)
    return new_times, new_mask, jnp.transpose(new_marks_t, (0, 2, 1))


# speedup vs baseline: 3.3402x; 1.1388x over previous
import jax
import jax.numpy as jnp
from jax.experimental import pallas as pl

B, M, D = 1024, 2048, 16
BR = 128
MQ = 512
NQ = M // MQ


def _body(times_ref, mask_ref, aux_ref, markt_ref, marks_ref,
          tout_ref, mout_ref, marksout_ref):
    m = mask_ref[...]
    cnt = jnp.sum(m.astype(jnp.int32), axis=1, keepdims=True)
    acc = aux_ref[:, 1:2] > 0.5
    canw = acc & (cnt < M)
    safe = jnp.minimum(cnt, M - 1)
    lanes = jax.lax.broadcasted_iota(jnp.int32, (BR, M), 1)
    sel = (lanes == safe) & canw
    tout_ref[...] = jnp.where(sel, aux_ref[:, 0:1], times_ref[...])
    mout_ref[...] = m | sel.astype(jnp.int8)
    i = pl.program_id(0)
    q = pl.program_id(1)
    mt = markt_ref[:, pl.ds(pl.multiple_of(i * BR, BR), BR)]
    markcol = jnp.transpose(mt, (1, 0))[:, :, None]
    qlanes = (jax.lax.broadcasted_iota(jnp.int32, (BR, MQ), 1)
              + q * MQ)
    qsel = (qlanes == safe) & canw
    marksout_ref[...] = jnp.where(qsel[:, None, :], markcol,
                                  marks_ref[...])


def kernel(times, mask, marks, t, mark, accepted):
    marks_t = jnp.transpose(marks, (0, 2, 1))
    mark_t = jnp.transpose(mark, (1, 0))
    mask8 = mask.astype(jnp.int8)
    aux = jnp.concatenate(
        [t[:, None], accepted.astype(jnp.float32)[:, None],
         jnp.zeros((B, 126), jnp.float32)], axis=1)

    grid = (B // BR, NQ)
    new_times, new_mask8, new_marks_t = pl.pallas_call(
        _body,
        grid=grid,
        in_specs=[
            pl.BlockSpec((BR, M), lambda i, q: (i, 0)),
            pl.BlockSpec((BR, M), lambda i, q: (i, 0)),
            pl.BlockSpec((BR, 128), lambda i, q: (i, 0)),
            pl.BlockSpec((D, B), lambda i, q: (0, 0)),
            pl.BlockSpec((BR, D, MQ), lambda i, q: (i, 0, q)),
        ],
        out_specs=[
            pl.BlockSpec((BR, M), lambda i, q: (i, 0)),
            pl.BlockSpec((BR, M), lambda i, q: (i, 0)),
            pl.BlockSpec((BR, D, MQ), lambda i, q: (i, 0, q)),
        ],
        out_shape=[
            jax.ShapeDtypeStruct((B, M), jnp.float32),
            jax.ShapeDtypeStruct((B, M), jnp.int8),
            jax.ShapeDtypeStruct((B, D, M), jnp.float32),
        ],
    )(times, mask8, aux, mark_t, marks_t)
    return (new_times, new_mask8.astype(jnp.bool_),
            jnp.transpose(new_marks_t, (0, 2, 1)))
